# baseline (speedup 1.0000x reference)
import functools

import jax
import jax.numpy as jnp
from jax import lax
from jax.experimental import pallas as pl
from jax.experimental.pallas import tpu as pltpu
from jax.experimental.pallas import tpu_sc as plsc

_CHUNK = 128
_K = 6
_L = 3


def _ceil_div(a, b):
    return (a + b - 1) // b


@functools.lru_cache(maxsize=None)
def _build(n_nodes, n_edges, hidden):
    info = plsc.get_sparse_core_info()
    nc, ns = info.num_cores, info.num_subcores
    nw = nc * ns

    assert n_edges % nw == 0 and (n_edges // nw) % 8 == 0
    e_per_w = n_edges // nw
    e_full = e_per_w // _CHUNK
    e_tail = e_per_w % _CHUNK

    n_chunks = n_nodes // _CHUNK
    n_tail = n_nodes % _CHUNK
    n_lo = n_chunks // nw
    n_extra = n_chunks % nw
    n_hi = n_lo + (1 if n_extra else 0)
    n_ld = max(n_hi * _CHUNK, 8)
    assert n_lo >= _K and e_full >= _K

    mesh = plsc.VectorSubcoreMesh(core_axis_name="c", subcore_axis_name="s")

    scratch = (
        [pltpu.VMEM((e_per_w,), jnp.int32),
         pltpu.VMEM((n_ld,), jnp.int32)]
        + [pltpu.VMEM((_CHUNK, hidden), jnp.float32) for _ in range(_K)]
        + [pltpu.SemaphoreType.DMA for _ in range(2 * _K)]
    )

    @functools.partial(
        pl.kernel,
        mesh=mesh,
        out_type=jax.ShapeDtypeStruct((n_nodes + n_edges, hidden), jnp.float32),
        scratch_types=scratch,
    )
    def k(node_ids, edge_ids, ntab, etab, out, e_ids, n_ids, *bufs_sems):
        bufs = bufs_sems[:_K]
        gsem = bufs_sems[_K:2 * _K]
        ssem = bufs_sems[2 * _K:]
        wid = lax.axis_index("s") * nc + lax.axis_index("c")

        def ring(ids_v, id_shift, tab, out_base, my_n, n_bound):
            def start_gather(p, pb):
                src = tab.at[ids_v.at[pl.ds(id_shift + p * _CHUNK, _CHUNK)]]
                pltpu.make_async_copy(src, bufs[pb], gsem[pb]).start()

            def wait_gather(b):
                pltpu.make_async_copy(
                    tab.at[ids_v.at[pl.ds(0, _CHUNK)]], bufs[b], gsem[b]).wait()

            def start_scatter(i, b):
                dst = out.at[pl.ds(out_base + i * _CHUNK, _CHUNK)]
                pltpu.make_async_copy(bufs[b], dst, ssem[b]).start()

            def wait_scatter(b):
                pltpu.make_async_copy(
                    bufs[b], out.at[pl.ds(0, _CHUNK)], ssem[b]).wait()

            for b0 in range(_L):
                start_gather(b0, b0)

            def body(m, carry):
                for b in range(_K):
                    i = m * _K + b
                    p = i + _L
                    pb = (b + _L) % _K

                    @pl.when(jnp.logical_and(p < my_n, p >= _K))
                    def _(pb=pb):
                        wait_scatter(pb)

                    @pl.when(p < my_n)
                    def _(p=p, pb=pb):
                        start_gather(p, pb)

                    @pl.when(i < my_n)
                    def _(i=i, b=b):
                        wait_gather(b)
                        start_scatter(i, b)

                return carry

            lax.fori_loop(0, _ceil_div(n_bound, _K), body, 0)
            for b in range(_K):
                wait_scatter(b)

        e_base = wid * e_per_w
        pltpu.sync_copy(edge_ids.at[pl.ds(e_base, e_per_w)], e_ids)
        ring(e_ids, 0, etab, n_nodes + e_base, e_full, e_full)

        if e_tail:
            off = e_full * _CHUNK
            src = etab.at[e_ids.at[pl.ds(off, e_tail)]]
            pltpu.async_copy(src, bufs[0].at[pl.ds(0, e_tail)], gsem[0]).wait()
            pltpu.sync_copy(bufs[0].at[pl.ds(0, e_tail)],
                            out.at[pl.ds(n_nodes + e_base + off, e_tail)])

        nbase_chunk = wid * n_lo + jnp.minimum(wid, n_extra)
        nbase_ids = nbase_chunk * _CHUNK
        ld_off = jnp.minimum(nbase_ids, n_nodes - n_ld)
        pltpu.sync_copy(node_ids.at[pl.ds(ld_off, n_ld)], n_ids)
        my_nn = jnp.where(wid < n_extra, n_hi, n_lo)
        ring(n_ids, nbase_ids - ld_off, ntab, nbase_ids, my_nn, n_hi)

        if n_tail:
            @pl.when(wid == nw - 1)
            def _():
                off = n_chunks * _CHUNK
                pltpu.sync_copy(node_ids.at[pl.ds(off, n_tail)],
                                n_ids.at[pl.ds(0, n_tail)])
                src = ntab.at[n_ids.at[pl.ds(0, n_tail)]]
                pltpu.async_copy(src, bufs[0].at[pl.ds(0, n_tail)],
                                 gsem[0]).wait()
                pltpu.sync_copy(bufs[0].at[pl.ds(0, n_tail)],
                                out.at[pl.ds(off, n_tail)])

    return k


def kernel(node_type_ids, edge_type_ids, node_type_table, edge_type_table):
    n_nodes = node_type_ids.shape[0]
    n_edges = edge_type_ids.shape[0]
    hidden = node_type_table.shape[1]
    k = _build(n_nodes, n_edges, hidden)
    return k(node_type_ids.astype(jnp.int32), edge_type_ids.astype(jnp.int32),
             node_type_table, edge_type_table)


# speedup vs baseline: 7.8288x; 7.8288x over previous
import functools

import jax
import jax.numpy as jnp
from jax import lax
from jax.experimental import pallas as pl
from jax.experimental.pallas import tpu as pltpu
from jax.experimental.pallas import tpu_sc as plsc

_C = 256
_K = 3
_LANES = 16


def _ceil_div(a, b):
    return (a + b - 1) // b


@functools.lru_cache(maxsize=None)
def _build(n_nodes, n_edges, hidden, n_nt, n_et):
    info = plsc.get_sparse_core_info()
    nc, ns = info.num_cores, info.num_subcores
    nw = nc * ns
    vecs = hidden // _LANES

    assert n_edges % nw == 0 and (n_edges // nw) % 8 == 0
    e_per_w = n_edges // nw
    e_full = e_per_w // _C
    e_tail = e_per_w % _C

    n_blocks = n_nodes // _C
    n_tail = n_nodes % _C
    n_lo = n_blocks // nw
    n_extra = n_blocks % nw
    n_hi = n_lo + (1 if n_extra else 0)
    n_ld = max(n_hi * _C, 8)
    assert n_lo >= _K and e_full >= _K

    mesh = plsc.VectorSubcoreMesh(core_axis_name="c", subcore_axis_name="s")

    scratch = (
        [pltpu.VMEM((e_per_w + _LANES,), jnp.int32),
         pltpu.VMEM((n_ld + _LANES,), jnp.int32),
         pltpu.VMEM((n_nt, hidden), jnp.float32),
         pltpu.VMEM((n_et, hidden), jnp.float32)]
        + [pltpu.VMEM((_C, hidden), jnp.float32) for _ in range(_K)]
        + [pltpu.SemaphoreType.DMA for _ in range(_K)]
    )

    @functools.partial(
        pl.kernel,
        mesh=mesh,
        out_type=jax.ShapeDtypeStruct((n_nodes + n_edges, hidden), jnp.float32),
        scratch_types=scratch,
    )
    def k(node_ids, edge_ids, ntab, etab, out,
          e_ids, n_ids, ntab_v, etab_v, *bufs_sems):
        bufs = bufs_sems[:_K]
        ssem = bufs_sems[_K:]
        wid = lax.axis_index("s") * nc + lax.axis_index("c")

        pltpu.sync_copy(ntab, ntab_v)
        pltpu.sync_copy(etab, etab_v)

        def build_group(ids_v, off, tab_v, buf, roff, g, width):
            idv = ids_v[pl.ds(off + g * _LANES, _LANES)]
            for u in range(width):
                rr = roff + g * _LANES + u
                tid = idv[u]
                for c in range(vecs):
                    sl = pl.ds(c * _LANES, _LANES)
                    buf[rr, sl] = tab_v[tid, sl]

        def build_rows(ids_v, off, tab_v, buf, rows):
            full, rem = divmod(rows, _LANES)

            def rb(g, carry):
                build_group(ids_v, off, tab_v, buf, 0, g, _LANES)
                return carry

            lax.fori_loop(0, full, rb, 0)
            if rem:
                build_group(ids_v, off, tab_v, buf, 0, full, rem)

        def wait_scatter(b):
            pltpu.make_async_copy(bufs[b], out.at[pl.ds(0, _C)], ssem[b]).wait()

        def ring(ids_v, id_shift, tab_v, out_base, my_n, n_bound):
            def body(m, carry):
                for b in range(_K):
                    i = m * _K + b

                    @pl.when(i < my_n)
                    def _(i=i, b=b):
                        @pl.when(i >= _K)
                        def _():
                            wait_scatter(b)

                        build_rows(ids_v, id_shift + i * _C, tab_v, bufs[b], _C)
                        dst = out.at[pl.ds(out_base + i * _C, _C)]
                        pltpu.make_async_copy(bufs[b], dst, ssem[b]).start()

                return carry

            lax.fori_loop(0, _ceil_div(n_bound, _K), body, 0)
            for b in range(_K):
                wait_scatter(b)

        e_base = wid * e_per_w
        pltpu.sync_copy(edge_ids.at[pl.ds(e_base, e_per_w)],
                        e_ids.at[pl.ds(0, e_per_w)])
        ring(e_ids, 0, etab_v, n_nodes + e_base, e_full, e_full)

        if e_tail:
            off = e_full * _C
            build_rows(e_ids, off, etab_v, bufs[0], e_tail)
            pltpu.sync_copy(bufs[0].at[pl.ds(0, e_tail)],
                            out.at[pl.ds(n_nodes + e_base + off, e_tail)])

        nbase_blk = wid * n_lo + jnp.minimum(wid, n_extra)
        nbase_ids = nbase_blk * _C
        ld_off = jnp.minimum(nbase_ids, n_nodes - n_ld)
        pltpu.sync_copy(node_ids.at[pl.ds(ld_off, n_ld)],
                        n_ids.at[pl.ds(0, n_ld)])
        my_nn = jnp.where(wid < n_extra, n_hi, n_lo)
        ring(n_ids, nbase_ids - ld_off, ntab_v, nbase_ids, my_nn, n_hi)

        if n_tail:
            @pl.when(wid == nw - 1)
            def _():
                off = n_blocks * _C
                pltpu.sync_copy(node_ids.at[pl.ds(off, n_tail)],
                                n_ids.at[pl.ds(0, n_tail)])
                build_rows(n_ids, 0, ntab_v, bufs[0], n_tail)
                pltpu.sync_copy(bufs[0].at[pl.ds(0, n_tail)],
                                out.at[pl.ds(off, n_tail)])

    return k


def kernel(node_type_ids, edge_type_ids, node_type_table, edge_type_table):
    n_nodes = node_type_ids.shape[0]
    n_edges = edge_type_ids.shape[0]
    hidden = node_type_table.shape[1]
    k = _build(n_nodes, n_edges, hidden,
               node_type_table.shape[0], edge_type_table.shape[0])
    return k(node_type_ids.astype(jnp.int32), edge_type_ids.astype(jnp.int32),
             node_type_table, edge_type_table)


# speedup vs baseline: 31.1320x; 3.9766x over previous
import functools

import jax
import jax.numpy as jnp
from jax import lax
from jax.experimental import pallas as pl
from jax.experimental.pallas import tpu as pltpu
from jax.experimental.pallas import tpu_sc as plsc

_CHUNK = 128
_K = 6
_L = 3


def _ceil_div(a, b):
    return (a + b - 1) // b


@functools.lru_cache(maxsize=None)
def _build(n_nodes, n_edges, hidden, n_nt, n_et):
    info = plsc.get_sparse_core_info()
    nc, ns = info.num_cores, info.num_subcores
    nw = nc * ns

    assert n_edges % nw == 0 and (n_edges // nw) % 8 == 0
    e_per_w = n_edges // nw
    e_full = e_per_w // _CHUNK
    e_tail = e_per_w % _CHUNK

    n_chunks = n_nodes // _CHUNK
    n_tail = n_nodes % _CHUNK
    n_lo = n_chunks // nw
    n_extra = n_chunks % nw
    n_hi = n_lo + (1 if n_extra else 0)
    n_ld = max(n_hi * _CHUNK, 8)
    assert n_lo >= _K and e_full >= _K

    mesh = plsc.VectorSubcoreMesh(core_axis_name="c", subcore_axis_name="s")

    scratch = (
        [pltpu.VMEM((e_per_w,), jnp.int32),
         pltpu.VMEM((n_ld,), jnp.int32),
         pltpu.VMEM_SHARED((n_nt, hidden), jnp.float32),
         pltpu.VMEM_SHARED((n_et, hidden), jnp.float32)]
        + [pltpu.VMEM((_CHUNK, hidden), jnp.float32) for _ in range(_K)]
        + [pltpu.SemaphoreType.DMA for _ in range(2 * _K)]
    )

    @functools.partial(
        pl.kernel,
        mesh=mesh,
        out_type=jax.ShapeDtypeStruct((n_nodes + n_edges, hidden), jnp.float32),
        scratch_types=scratch,
    )
    def k(node_ids, edge_ids, ntab, etab, out,
          e_ids, n_ids, ntab_sp, etab_sp, *bufs_sems):
        bufs = bufs_sems[:_K]
        gsem = bufs_sems[_K:2 * _K]
        ssem = bufs_sems[2 * _K:]
        wid = lax.axis_index("s") * nc + lax.axis_index("c")

        @pl.when(lax.axis_index("s") == 0)
        def _():
            pltpu.sync_copy(ntab, ntab_sp)
            pltpu.sync_copy(etab, etab_sp)

        plsc.subcore_barrier()

        def ring(ids_v, id_shift, tab_sp, out_base, my_n, n_bound):
            def start_gather(p, pb):
                src = tab_sp.at[ids_v.at[pl.ds(id_shift + p * _CHUNK, _CHUNK)]]
                pltpu.make_async_copy(src, bufs[pb], gsem[pb]).start()

            def wait_gather(b):
                pltpu.make_async_copy(
                    tab_sp.at[ids_v.at[pl.ds(0, _CHUNK)]],
                    bufs[b], gsem[b]).wait()

            def start_scatter(i, b):
                dst = out.at[pl.ds(out_base + i * _CHUNK, _CHUNK)]
                pltpu.make_async_copy(bufs[b], dst, ssem[b]).start()

            def wait_scatter(b):
                pltpu.make_async_copy(
                    bufs[b], out.at[pl.ds(0, _CHUNK)], ssem[b]).wait()

            for b0 in range(_L):
                start_gather(b0, b0)

            def body(m, carry):
                for b in range(_K):
                    i = m * _K + b
                    p = i + _L
                    pb = (b + _L) % _K

                    @pl.when(jnp.logical_and(p < my_n, p >= _K))
                    def _(pb=pb):
                        wait_scatter(pb)

                    @pl.when(p < my_n)
                    def _(p=p, pb=pb):
                        start_gather(p, pb)

                    @pl.when(i < my_n)
                    def _(i=i, b=b):
                        wait_gather(b)
                        start_scatter(i, b)

                return carry

            lax.fori_loop(0, _ceil_div(n_bound, _K), body, 0)
            for b in range(_K):
                wait_scatter(b)

        e_base = wid * e_per_w
        pltpu.sync_copy(edge_ids.at[pl.ds(e_base, e_per_w)], e_ids)
        ring(e_ids, 0, etab_sp, n_nodes + e_base, e_full, e_full)

        if e_tail:
            off = e_full * _CHUNK
            src = etab_sp.at[e_ids.at[pl.ds(off, e_tail)]]
            pltpu.async_copy(src, bufs[0].at[pl.ds(0, e_tail)], gsem[0]).wait()
            pltpu.sync_copy(bufs[0].at[pl.ds(0, e_tail)],
                            out.at[pl.ds(n_nodes + e_base + off, e_tail)])

        nbase_chunk = wid * n_lo + jnp.minimum(wid, n_extra)
        nbase_ids = nbase_chunk * _CHUNK
        ld_off = jnp.minimum(nbase_ids, n_nodes - n_ld)
        pltpu.sync_copy(node_ids.at[pl.ds(ld_off, n_ld)],
                        n_ids.at[pl.ds(0, n_ld)])
        my_nn = jnp.where(wid < n_extra, n_hi, n_lo)
        ring(n_ids, nbase_ids - ld_off, ntab_sp, nbase_ids, my_nn, n_hi)

        if n_tail:
            @pl.when(wid == nw - 1)
            def _():
                off = n_chunks * _CHUNK
                pltpu.sync_copy(node_ids.at[pl.ds(off, n_tail)],
                                n_ids.at[pl.ds(0, n_tail)])
                src = ntab_sp.at[n_ids.at[pl.ds(0, n_tail)]]
                pltpu.async_copy(src, bufs[0].at[pl.ds(0, n_tail)],
                                 gsem[0]).wait()
                pltpu.sync_copy(bufs[0].at[pl.ds(0, n_tail)],
                                out.at[pl.ds(off, n_tail)])

    return k


def kernel(node_type_ids, edge_type_ids, node_type_table, edge_type_table):
    n_nodes = node_type_ids.shape[0]
    n_edges = edge_type_ids.shape[0]
    hidden = node_type_table.shape[1]
    k = _build(n_nodes, n_edges, hidden,
               node_type_table.shape[0], edge_type_table.shape[0])
    return k(node_type_ids.astype(jnp.int32), edge_type_ids.astype(jnp.int32),
             node_type_table, edge_type_table)
